# baseline (speedup 1.0000x reference)
import functools

import jax
import jax.numpy as jnp
from jax import lax
from jax.experimental import pallas as pl
from jax.experimental.pallas import tpu as pltpu
from jax.experimental.pallas import tpu_sc as plsc


@functools.lru_cache(maxsize=None)
def _build_gather(per_w: int, chunk: int, d: int, nc: int, ns: int):
    nchunk = per_w // chunk
    nw = nc * ns
    n_pad = nw * per_w
    mesh = plsc.VectorSubcoreMesh(core_axis_name="c", subcore_axis_name="s")

    @functools.partial(
        pl.kernel,
        mesh=mesh,
        out_type=jax.ShapeDtypeStruct((n_pad, d), jnp.float32),
        scratch_types=[
            pltpu.VMEM((nchunk, chunk), jnp.int32),
            pltpu.VMEM((chunk, d), jnp.float32),
            pltpu.SemaphoreType.DMA,
        ],
        compiler_params=pltpu.CompilerParams(use_tc_tiling_on_sc=False),
    )
    def gather_kernel(x_hbm, table_hbm, out_hbm, idx_v, rows_v, sem):
        wid = lax.axis_index("s") * nc + lax.axis_index("c")
        base = wid * per_w
        pltpu.sync_copy(x_hbm.at[wid], idx_v)

        def body(j, carry):
            pltpu.async_copy(table_hbm.at[idx_v.at[j]], rows_v, sem).wait()
            pltpu.sync_copy(rows_v, out_hbm.at[pl.ds(base + j * chunk, chunk)])
            return carry

        lax.fori_loop(0, nchunk, body, 0)

    return gather_kernel


def kernel(x, table):
    n = x.shape[0]
    d = table.shape[1]
    info = plsc.get_sparse_core_info()
    nc, ns = info.num_cores, info.num_subcores
    nw = nc * ns
    chunk = 128
    per_w = -(-n // (nw * chunk)) * chunk
    n_pad = per_w * nw

    xi = x.astype(jnp.int32)
    x_pad = jnp.zeros((n_pad,), jnp.int32).at[:n].set(xi)
    x_blocks = x_pad.reshape(nw, per_w // chunk, chunk)
    out = _build_gather(per_w, chunk, d, nc, ns)(x_blocks, table)
    return out[:n]


# speedup vs baseline: 1.0440x; 1.0440x over previous
import functools

import jax
import jax.numpy as jnp
from jax import lax
from jax.experimental import pallas as pl
from jax.experimental.pallas import tpu as pltpu
from jax.experimental.pallas import tpu_sc as plsc


@functools.lru_cache(maxsize=None)
def _build_gather(per_w: int, chunk: int, d: int, nc: int, ns: int):
    nchunk = per_w // chunk
    nw = nc * ns
    n_pad = nw * per_w
    mesh = plsc.VectorSubcoreMesh(core_axis_name="c", subcore_axis_name="s")

    @functools.partial(
        pl.kernel,
        mesh=mesh,
        out_type=jax.ShapeDtypeStruct((n_pad, d), jnp.float32),
        scratch_types=[
            pltpu.VMEM((nchunk, chunk), jnp.int32),
            pltpu.VMEM((chunk, d), jnp.float32),
            pltpu.SemaphoreType.DMA,
        ],
        compiler_params=pltpu.CompilerParams(use_tc_tiling_on_sc=False),
    )
    def gather_kernel(x_hbm, table_hbm, out_hbm, idx_v, rows_v, sem):
        wid = lax.axis_index("s") * nc + lax.axis_index("c")
        base = wid * per_w
        pltpu.sync_copy(x_hbm.at[wid], idx_v)

        def body(j, carry):
            pltpu.async_copy(table_hbm.at[idx_v.at[j]], rows_v, sem).wait()
            pltpu.sync_copy(rows_v, out_hbm.at[pl.ds(base + j * chunk, chunk)])
            return carry

        lax.fori_loop(0, nchunk, body, 0)

    return gather_kernel


def kernel(x, table):
    n = x.shape[0]
    d = table.shape[1]
    info = plsc.get_sparse_core_info()
    nc, ns = info.num_cores, info.num_subcores
    nw = nc * ns
    chunk = 800
    per_w = -(-n // (nw * chunk)) * chunk
    n_pad = per_w * nw

    xi = x.astype(jnp.int32)
    x_pad = jnp.zeros((n_pad,), jnp.int32).at[:n].set(xi)
    x_blocks = x_pad.reshape(nw, per_w // chunk, chunk)
    out = _build_gather(per_w, chunk, d, nc, ns)(x_blocks, table)
    return out[:n]


# speedup vs baseline: 1.7173x; 1.6449x over previous
import functools

import jax
import jax.numpy as jnp
from jax import lax
from jax.experimental import pallas as pl
from jax.experimental.pallas import tpu as pltpu
from jax.experimental.pallas import tpu_sc as plsc


@functools.lru_cache(maxsize=None)
def _build_gather(n: int, per_w: int, chunk: int, d: int, nc: int, ns: int):
    nchunk = per_w // chunk
    nbuf = 4
    look = 2
    mesh = plsc.VectorSubcoreMesh(core_axis_name="c", subcore_axis_name="s")

    scratch = [pltpu.VMEM((nchunk, chunk), jnp.int32)]
    scratch += [pltpu.VMEM((chunk, d), jnp.float32) for _ in range(nbuf)]
    scratch += [pltpu.SemaphoreType.DMA for _ in range(2 * nbuf)]

    @functools.partial(
        pl.kernel,
        mesh=mesh,
        out_type=jax.ShapeDtypeStruct((n, d), jnp.float32),
        scratch_types=scratch,
        compiler_params=pltpu.CompilerParams(use_tc_tiling_on_sc=False),
    )
    def gather_kernel(x_hbm, table_hbm, out_hbm, idx_v, *rest):
        bufs = rest[:nbuf]
        gsem = rest[nbuf:2 * nbuf]
        wsem = rest[2 * nbuf:3 * nbuf]
        wid = lax.axis_index("s") * nc + lax.axis_index("c")
        base = wid * per_w
        pltpu.sync_copy(x_hbm.at[wid], idx_v)

        def valid(j):
            return base + j * chunk < n

        def gather_desc(j):
            return pltpu.make_async_copy(
                table_hbm.at[idx_v.at[j]], bufs[j % nbuf], gsem[j % nbuf])

        def write_desc(j):
            return pltpu.make_async_copy(
                bufs[j % nbuf],
                out_hbm.at[pl.ds(base + j * chunk, chunk)],
                wsem[j % nbuf])

        def when_valid(j, fn):
            pl.when(valid(j))(fn)

        for j in range(look):
            gather_desc(j).start()

        for j in range(nchunk):
            jj = j + look
            if jj < nchunk:
                if jj - nbuf >= 0:
                    when_valid(jj - nbuf, write_desc(jj - nbuf).wait)
                when_valid(jj, gather_desc(jj).start)
            if j < look:
                gather_desc(j).wait()
            else:
                when_valid(j, gather_desc(j).wait)
            when_valid(j, write_desc(j).start)

        for j in range(max(0, nchunk - nbuf), nchunk):
            when_valid(j, write_desc(j).wait)

    return gather_kernel


def kernel(x, table):
    n = x.shape[0]
    vocab, d = table.shape
    info = plsc.get_sparse_core_info()
    nc, ns = info.num_cores, info.num_subcores
    nw = nc * ns
    chunk = 400
    per_w = -(-n // (nw * chunk)) * chunk
    n_pad = per_w * nw

    xi = x.astype(jnp.int32)
    pad = jnp.arange(n_pad - n, dtype=jnp.int32) % vocab
    x_blocks = jnp.concatenate([xi, pad]).reshape(nw, per_w // chunk, chunk)
    return _build_gather(n, per_w, chunk, d, nc, ns)(x_blocks, table)


# speedup vs baseline: 1.7184x; 1.0006x over previous
import functools

import jax
import jax.numpy as jnp
from jax import lax
from jax.experimental import pallas as pl
from jax.experimental.pallas import tpu as pltpu
from jax.experimental.pallas import tpu_sc as plsc


@functools.lru_cache(maxsize=None)
def _build_gather(n: int, per_w: int, chunk: int, d: int, nc: int, ns: int):
    nchunk = per_w // chunk
    nbuf = 4
    look = 3
    mesh = plsc.VectorSubcoreMesh(core_axis_name="c", subcore_axis_name="s")

    scratch = [pltpu.VMEM((nchunk, chunk), jnp.int32)]
    scratch += [pltpu.VMEM((chunk, d), jnp.float32) for _ in range(nbuf)]
    scratch += [pltpu.SemaphoreType.DMA for _ in range(2 * nbuf)]

    @functools.partial(
        pl.kernel,
        mesh=mesh,
        out_type=jax.ShapeDtypeStruct((n, d), jnp.float32),
        scratch_types=scratch,
        compiler_params=pltpu.CompilerParams(use_tc_tiling_on_sc=False),
    )
    def gather_kernel(x_hbm, table_hbm, out_hbm, idx_v, *rest):
        bufs = rest[:nbuf]
        gsem = rest[nbuf:2 * nbuf]
        wsem = rest[2 * nbuf:3 * nbuf]
        wid = lax.axis_index("s") * nc + lax.axis_index("c")
        base = wid * per_w
        pltpu.sync_copy(x_hbm.at[wid], idx_v)

        def valid(j):
            return base + j * chunk < n

        def gather_desc(j):
            return pltpu.make_async_copy(
                table_hbm.at[idx_v.at[j]], bufs[j % nbuf], gsem[j % nbuf])

        def write_desc(j):
            return pltpu.make_async_copy(
                bufs[j % nbuf],
                out_hbm.at[pl.ds(base + j * chunk, chunk)],
                wsem[j % nbuf])

        def when_valid(j, fn):
            pl.when(valid(j))(fn)

        for j in range(look):
            gather_desc(j).start()

        for j in range(nchunk):
            jj = j + look
            if jj < nchunk:
                if jj - nbuf >= 0:
                    when_valid(jj - nbuf, write_desc(jj - nbuf).wait)
                when_valid(jj, gather_desc(jj).start)
            if j < look:
                gather_desc(j).wait()
            else:
                when_valid(j, gather_desc(j).wait)
            when_valid(j, write_desc(j).start)

        for j in range(max(0, nchunk - nbuf), nchunk):
            when_valid(j, write_desc(j).wait)

    return gather_kernel


def kernel(x, table):
    n = x.shape[0]
    vocab, d = table.shape
    info = plsc.get_sparse_core_info()
    nc, ns = info.num_cores, info.num_subcores
    nw = nc * ns
    chunk = 400
    per_w = -(-n // (nw * chunk)) * chunk
    n_pad = per_w * nw

    xi = x.astype(jnp.int32)
    pad = jnp.arange(n_pad - n, dtype=jnp.int32) % vocab
    x_blocks = jnp.concatenate([xi, pad]).reshape(nw, per_w // chunk, chunk)
    return _build_gather(n, per_w, chunk, d, nc, ns)(x_blocks, table)


# speedup vs baseline: 2.0731x; 1.2064x over previous
import functools

import jax
import jax.numpy as jnp
from jax import lax
from jax.experimental import pallas as pl
from jax.experimental.pallas import tpu as pltpu
from jax.experimental.pallas import tpu_sc as plsc


@functools.lru_cache(maxsize=None)
def _build_gather(n: int, per_w: int, chunk: int, dp: int, nc: int, ns: int):
    nchunk = per_w // chunk
    nbuf = 2
    look = 1
    mesh = plsc.VectorSubcoreMesh(core_axis_name="c", subcore_axis_name="s")

    scratch = [pltpu.VMEM((per_w,), jnp.int32)]
    scratch += [pltpu.VMEM((chunk, dp), jnp.float32) for _ in range(nbuf)]
    scratch += [pltpu.SemaphoreType.DMA for _ in range(2 * nbuf)]

    @functools.partial(
        pl.kernel,
        mesh=mesh,
        out_type=jax.ShapeDtypeStruct((n, dp), jnp.float32),
        scratch_types=scratch,
        compiler_params=pltpu.CompilerParams(use_tc_tiling_on_sc=True),
    )
    def gather_kernel(x_hbm, table_hbm, out_hbm, idx_v, *rest):
        bufs = rest[:nbuf]
        gsem = rest[nbuf:2 * nbuf]
        wsem = rest[2 * nbuf:3 * nbuf]
        wid = lax.axis_index("s") * nc + lax.axis_index("c")
        base = wid * per_w
        pltpu.sync_copy(x_hbm.at[pl.ds(base, per_w)], idx_v)

        def valid(j):
            return base + j * chunk < n

        def gather_desc(j):
            return pltpu.make_async_copy(
                table_hbm.at[idx_v.at[pl.ds(j * chunk, chunk)]],
                bufs[j % nbuf], gsem[j % nbuf])

        def write_desc(j):
            return pltpu.make_async_copy(
                bufs[j % nbuf],
                out_hbm.at[pl.ds(base + j * chunk, chunk)],
                wsem[j % nbuf])

        def when_valid(j, fn):
            pl.when(valid(j))(fn)

        for j in range(look):
            gather_desc(j).start()

        for j in range(nchunk):
            jj = j + look
            if jj < nchunk:
                if jj - nbuf >= 0:
                    when_valid(jj - nbuf, write_desc(jj - nbuf).wait)
                when_valid(jj, gather_desc(jj).start)
            if j < look:
                gather_desc(j).wait()
            else:
                when_valid(j, gather_desc(j).wait)
            when_valid(j, write_desc(j).start)

        for j in range(max(0, nchunk - nbuf), nchunk):
            when_valid(j, write_desc(j).wait)

    return gather_kernel


def kernel(x, table):
    n = x.shape[0]
    vocab, d = table.shape
    dp = 128
    info = plsc.get_sparse_core_info()
    nc, ns = info.num_cores, info.num_subcores
    nw = nc * ns
    chunk = 400
    per_w = -(-n // (nw * chunk)) * chunk
    n_pad = per_w * nw

    xi = x.astype(jnp.int32)
    pad = jnp.arange(n_pad - n, dtype=jnp.int32) % vocab
    x_pad = jnp.concatenate([xi, pad])
    table_p = jnp.pad(table, ((0, 0), (0, dp - d)))
    out = _build_gather(n, per_w, chunk, dp, nc, ns)(x_pad, table_p)
    return out[:, :d]


# speedup vs baseline: 2.0813x; 1.0039x over previous
import functools

import jax
import jax.numpy as jnp
from jax import lax
from jax.experimental import pallas as pl
from jax.experimental.pallas import tpu as pltpu
from jax.experimental.pallas import tpu_sc as plsc


@functools.lru_cache(maxsize=None)
def _build_gather(n: int, per_w: int, chunk: int, dp: int, nc: int, ns: int):
    nchunk = per_w // chunk
    nbuf = 4
    look = 2
    mesh = plsc.VectorSubcoreMesh(core_axis_name="c", subcore_axis_name="s")

    scratch = [pltpu.VMEM((per_w,), jnp.int32)]
    scratch += [pltpu.VMEM((chunk, dp), jnp.float32) for _ in range(nbuf)]
    scratch += [pltpu.SemaphoreType.DMA for _ in range(2 * nbuf)]

    @functools.partial(
        pl.kernel,
        mesh=mesh,
        out_type=jax.ShapeDtypeStruct((n, dp), jnp.float32),
        scratch_types=scratch,
        compiler_params=pltpu.CompilerParams(use_tc_tiling_on_sc=True),
    )
    def gather_kernel(x_hbm, table_hbm, out_hbm, idx_v, *rest):
        bufs = rest[:nbuf]
        gsem = rest[nbuf:2 * nbuf]
        wsem = rest[2 * nbuf:3 * nbuf]
        wid = lax.axis_index("s") * nc + lax.axis_index("c")
        base = wid * per_w
        pltpu.sync_copy(x_hbm.at[pl.ds(base, per_w)], idx_v)

        def valid(j):
            return base + j * chunk < n

        def gather_desc(j):
            return pltpu.make_async_copy(
                table_hbm.at[idx_v.at[pl.ds(j * chunk, chunk)]],
                bufs[j % nbuf], gsem[j % nbuf])

        def write_desc(j):
            return pltpu.make_async_copy(
                bufs[j % nbuf],
                out_hbm.at[pl.ds(base + j * chunk, chunk)],
                wsem[j % nbuf])

        def when_valid(j, fn):
            pl.when(valid(j))(fn)

        for j in range(look):
            gather_desc(j).start()

        for j in range(nchunk):
            jj = j + look
            if jj < nchunk:
                if jj - nbuf >= 0:
                    when_valid(jj - nbuf, write_desc(jj - nbuf).wait)
                when_valid(jj, gather_desc(jj).start)
            if j < look:
                gather_desc(j).wait()
            else:
                when_valid(j, gather_desc(j).wait)
            when_valid(j, write_desc(j).start)

        for j in range(max(0, nchunk - nbuf), nchunk):
            when_valid(j, write_desc(j).wait)

    return gather_kernel


def kernel(x, table):
    n = x.shape[0]
    vocab, d = table.shape
    dp = 128
    info = plsc.get_sparse_core_info()
    nc, ns = info.num_cores, info.num_subcores
    nw = nc * ns
    chunk = 200
    per_w = -(-n // (nw * chunk)) * chunk
    n_pad = per_w * nw

    xi = x.astype(jnp.int32)
    pad = jnp.arange(n_pad - n, dtype=jnp.int32) % vocab
    x_pad = jnp.concatenate([xi, pad])
    table_p = jnp.pad(table, ((0, 0), (0, dp - d)))
    out = _build_gather(n, per_w, chunk, dp, nc, ns)(x_pad, table_p)
    return out[:, :d]
